# baseline (speedup 1.0000x reference)
import functools

import jax
import jax.numpy as jnp
from jax import lax
from jax.experimental import pallas as pl
from jax.experimental.pallas import tpu as pltpu
from jax.experimental.pallas import tpu_sc as plsc

N = 10000
H = 256
A = 128
NC = 2
NS = 16
BATCH = 128
NT = N + 112
STRIPE = NT // NS
EPAD = 163840
NB = EPAD // (NS * BATCH)
NBD = EPAD // (NC * NS * BATCH)
BN = 1000


def _zero_rows(buf, rows, cols):
    def zrow(r, _):
        for k in range(cols // 16):
            buf[r, pl.ds(k * 16, 16)] = jnp.zeros((16,), jnp.float32)
        return 0
    lax.fori_loop(0, rows, zrow, 0)


def _zero_acc_stripe(acc, zbuf, base):
    full, rem = STRIPE // 128, STRIPE % 128
    for k in range(full):
        pltpu.sync_copy(zbuf, acc.at[pl.ds(base + k * 128, 128)])
    if rem:
        pltpu.sync_copy(zbuf.at[pl.ds(0, rem)],
                        acc.at[pl.ds(base + full * 128, rem)])


def _deg_body(dstd, out, dst_v, e0, zbuf, acc, semd):
    c = lax.axis_index("c")
    s = lax.axis_index("s")
    w = c * NS + s
    pltpu.sync_copy(dstd.at[w], dst_v)
    _zero_rows(zbuf, 128, 16)
    v = jnp.where(lax.iota(jnp.int32, 16) == 0, jnp.float32(1), jnp.float32(0))

    def erow(r, _):
        e0[r, :] = v
        return 0
    lax.fori_loop(0, 128, erow, 0)
    _zero_acc_stripe(acc, zbuf, s * STRIPE)
    plsc.subcore_barrier()

    def body(j, _):
        pltpu.sync_copy(e0, acc.at[dst_v.at[j]], add=True)
        return 0
    lax.fori_loop(0, NBD, body, 0)
    plsc.subcore_barrier()
    pltpu.sync_copy(acc.at[pl.ds(s * STRIPE, STRIPE)],
                    out.at[pl.ds(c * NT + s * STRIPE, STRIPE)])


def _deg_counts(dstd):
    return pl.kernel(
        _deg_body,
        out_type=jax.ShapeDtypeStruct((NC * NT, 16), jnp.float32),
        mesh=plsc.VectorSubcoreMesh(core_axis_name="c", subcore_axis_name="s"),
        scratch_types=[
            pltpu.VMEM((NBD, BATCH), jnp.int32),
            pltpu.VMEM((128, 16), jnp.float32),
            pltpu.VMEM((128, 16), jnp.float32),
            pltpu.VMEM_SHARED((NT, 16), jnp.float32),
            pltpu.SemaphoreType.DMA,
        ],
    )(dstd)


CH = 40


def _scatter_body(gflat, srcp, dstp, out, src_v, dst_v, buf0, buf1, acc,
                  sem0, sem1, *, nb):
    c = lax.axis_index("c")
    s = lax.axis_index("s")
    w = c * NS + s
    _zero_rows(buf0, 128, BATCH)
    _zero_acc_stripe(acc, buf0, s * STRIPE)
    plsc.subcore_barrier()

    def step(b, buf, sem, start_next):
        pltpu.make_async_copy(gflat.at[src_v.at[b]], buf, sem).wait()
        pltpu.sync_copy(buf, acc.at[dst_v.at[b]], add=True)
        if start_next:
            pltpu.async_copy(gflat.at[src_v.at[b + 2]], buf, sem)

    def chunk(q, _):
        pltpu.sync_copy(srcp.at[w, pl.ds(q * CH, CH)], src_v)
        pltpu.sync_copy(dstp.at[w, pl.ds(q * CH, CH)], dst_v)
        pltpu.async_copy(gflat.at[src_v.at[0]], buf0, sem0)
        pltpu.async_copy(gflat.at[src_v.at[1]], buf1, sem1)

        def body(j2, _):
            step(2 * j2, buf0, sem0, True)
            step(2 * j2 + 1, buf1, sem1, True)
            return 0
        lax.fori_loop(0, CH // 2 - 1, body, 0)
        step(CH - 2, buf0, sem0, False)
        step(CH - 1, buf1, sem1, False)
        return 0
    lax.fori_loop(0, nb // CH, chunk, 0)
    plsc.subcore_barrier()
    pltpu.sync_copy(acc.at[pl.ds(s * STRIPE, STRIPE)],
                    out.at[pl.ds(c * NT + s * STRIPE, STRIPE)])


def _edge_scatter(gflat, srcp, dstp, nb):
    return pl.kernel(
        functools.partial(_scatter_body, nb=nb),
        out_type=jax.ShapeDtypeStruct((NC * NT, BATCH), jnp.float32),
        mesh=plsc.VectorSubcoreMesh(core_axis_name="c", subcore_axis_name="s"),
        scratch_types=[
            pltpu.VMEM((CH, BATCH), jnp.int32),
            pltpu.VMEM((CH, BATCH), jnp.int32),
            pltpu.VMEM((BATCH, BATCH), jnp.float32),
            pltpu.VMEM((BATCH, BATCH), jnp.float32),
            pltpu.VMEM_SHARED((NT, BATCH), jnp.float32),
            pltpu.SemaphoreType.DMA,
            pltpu.SemaphoreType.DMA,
        ],
    )(gflat, srcp, dstp)


def _dinv_of(deg_ref):
    dvec = deg_ref[0, :, 0:1] + deg_ref[1, :, 0:1] + jnp.float32(1)
    return lax.rsqrt(dvec)


def _t0_body(x_ref, w_ref, deg_ref, g_ref):
    dinv = _dinv_of(deg_ref)
    p = jnp.dot(x_ref[...], w_ref[...], preferred_element_type=jnp.float32)
    g = dinv * p
    hc = g.shape[1] // 2
    g_ref[0] = g[:, :hc]
    g_ref[1] = g[:, hc:]


def _mid_body(s_ref, g_ref, deg_ref, b_ref, w_ref, h_ref, gn_ref, *, split):
    dinv = _dinv_of(deg_ref)
    t = jnp.concatenate([s_ref[0] + g_ref[0], s_ref[1] + g_ref[1]], axis=1)
    h = jnp.tanh(dinv * t + b_ref[...])
    h_ref[...] = h
    p = jnp.dot(h, w_ref[...], preferred_element_type=jnp.float32)
    gn = dinv * p
    if split:
        hc = gn.shape[1] // 2
        gn_ref[0] = gn[:, :hc]
        gn_ref[1] = gn[:, hc:]
    else:
        gn_ref[...] = gn


def _fin_body(s_ref, g_ref, deg_ref, b_ref, o_ref):
    dinv = _dinv_of(deg_ref)
    t = s_ref[0] + s_ref[1] + g_ref[...]
    o_ref[...] = jnp.tanh(dinv * t + b_ref[...])


def _row_blk(shape):
    return pl.BlockSpec((1,) * (len(shape) - 2) + (BN, shape[-1]),
                        lambda i: (0,) * (len(shape) - 2) + (i, 0))


def _full_blk(shape):
    return pl.BlockSpec(shape, lambda i: (0,) * len(shape))


def _t0(x, w0, deg2):
    return pl.pallas_call(
        _t0_body,
        grid=(N // BN,),
        in_specs=[_row_blk((N, H)), _full_blk((H, H)),
                  pl.BlockSpec((2, BN, 16), lambda i: (0, i, 0))],
        out_specs=pl.BlockSpec((2, BN, H // 2), lambda i: (0, i, 0)),
        out_shape=jax.ShapeDtypeStruct((2, N, H // 2), jnp.float32),
    )(x, w0, deg2)


def _tmid(s2, g2, deg2, b, w, split):
    k = w.shape[1]
    if split:
        gn_spec = pl.BlockSpec((2, BN, k // 2), lambda i: (0, i, 0))
        gn_shape = jax.ShapeDtypeStruct((2, N, k // 2), jnp.float32)
    else:
        gn_spec = _row_blk((N, k))
        gn_shape = jax.ShapeDtypeStruct((N, k), jnp.float32)
    return pl.pallas_call(
        functools.partial(_mid_body, split=split),
        grid=(N // BN,),
        in_specs=[pl.BlockSpec((2, BN, H // 2), lambda i: (0, i, 0)),
                  pl.BlockSpec((2, BN, H // 2), lambda i: (0, i, 0)),
                  pl.BlockSpec((2, BN, 16), lambda i: (0, i, 0)),
                  _full_blk((1, H)), _full_blk((H, k))],
        out_specs=[_row_blk((N, H)), gn_spec],
        out_shape=[jax.ShapeDtypeStruct((N, H), jnp.float32), gn_shape],
    )(s2, g2, deg2, b, w)


def _tfin(s2, g2, deg2, b):
    return pl.pallas_call(
        _fin_body,
        grid=(N // BN,),
        in_specs=[pl.BlockSpec((2, BN, A), lambda i: (0, i, 0)),
                  _row_blk((N, A)),
                  pl.BlockSpec((2, BN, 16), lambda i: (0, i, 0)),
                  _full_blk((1, A))],
        out_specs=_row_blk((N, A)),
        out_shape=jax.ShapeDtypeStruct((N, A), jnp.float32),
    )(s2, g2, deg2, b)


def kernel(x, edge_index, W0, b0, W1, b1, W2, b2):
    e = edge_index.shape[1]
    src, dst = edge_index[0], edge_index[1]
    padidx = jnp.arange(EPAD - e, dtype=jnp.int32) % 16
    srcf = jnp.concatenate([src, padidx])
    dstf = jnp.concatenate([dst, N + padidx])
    src01 = jnp.stack([srcf, srcf + N]).reshape(NC * NS, NB, BATCH)
    dst01 = jnp.stack([dstf, dstf]).reshape(NC * NS, NB, BATCH)
    src2 = srcf.reshape(NC * NS, NBD, BATCH)
    dst2 = dstf.reshape(NC * NS, NBD, BATCH)

    deg2 = _deg_counts(dst2).reshape(NC, NT, 16)

    def layer_scatter01(gsplit):
        sflat = _edge_scatter(gsplit.reshape(NC * N, BATCH), src01, dst01, NB)
        return sflat.reshape(NC, NT, BATCH)

    g0 = _t0(x, W0, deg2)
    s0 = layer_scatter01(g0)
    h1, g1 = _tmid(s0, g0, deg2, b0.reshape(1, H), W1, split=True)
    s1 = layer_scatter01(g1)
    h2, g2 = _tmid(s1, g1, deg2, b1.reshape(1, H), W2, split=False)
    s2 = _edge_scatter(g2, src2, dst2, NBD).reshape(NC, NT, A)
    out = _tfin(s2, g2, deg2, b2.reshape(1, A))
    return (out, h1, h2)


# speedup vs baseline: 1.0121x; 1.0121x over previous
import functools

import jax
import jax.numpy as jnp
from jax import lax
from jax.experimental import pallas as pl
from jax.experimental.pallas import tpu as pltpu
from jax.experimental.pallas import tpu_sc as plsc

N = 10000
H = 256
A = 128
NC = 2
NS = 16
BATCH = 128
NT = N + 112
STRIPE = NT // NS
EPAD = 163840
NB = EPAD // (NS * BATCH)
NBD = EPAD // (NC * NS * BATCH)
BN = 2000


def _zero_rows(buf, rows, cols):
    def zrow(r, _):
        for k in range(cols // 16):
            buf[r, pl.ds(k * 16, 16)] = jnp.zeros((16,), jnp.float32)
        return 0
    lax.fori_loop(0, rows, zrow, 0)


def _zero_acc_stripe(acc, zbuf, base):
    full, rem = STRIPE // 128, STRIPE % 128
    for k in range(full):
        pltpu.sync_copy(zbuf, acc.at[pl.ds(base + k * 128, 128)])
    if rem:
        pltpu.sync_copy(zbuf.at[pl.ds(0, rem)],
                        acc.at[pl.ds(base + full * 128, rem)])


def _deg_body(dstd, out, dst_v, e0, zbuf, acc, semd):
    c = lax.axis_index("c")
    s = lax.axis_index("s")
    w = c * NS + s
    pltpu.sync_copy(dstd.at[w], dst_v)
    _zero_rows(zbuf, 128, 16)
    v = jnp.where(lax.iota(jnp.int32, 16) == 0, jnp.float32(1), jnp.float32(0))

    def erow(r, _):
        e0[r, :] = v
        return 0
    lax.fori_loop(0, 128, erow, 0)
    _zero_acc_stripe(acc, zbuf, s * STRIPE)
    plsc.subcore_barrier()

    def body(j, _):
        pltpu.sync_copy(e0, acc.at[dst_v.at[j]], add=True)
        return 0
    lax.fori_loop(0, NBD, body, 0)
    plsc.subcore_barrier()
    pltpu.sync_copy(acc.at[pl.ds(s * STRIPE, STRIPE)],
                    out.at[pl.ds(c * NT + s * STRIPE, STRIPE)])


def _deg_counts(dstd):
    return pl.kernel(
        _deg_body,
        out_type=jax.ShapeDtypeStruct((NC * NT, 16), jnp.float32),
        mesh=plsc.VectorSubcoreMesh(core_axis_name="c", subcore_axis_name="s"),
        scratch_types=[
            pltpu.VMEM((NBD, BATCH), jnp.int32),
            pltpu.VMEM((128, 16), jnp.float32),
            pltpu.VMEM((128, 16), jnp.float32),
            pltpu.VMEM_SHARED((NT, 16), jnp.float32),
            pltpu.SemaphoreType.DMA,
        ],
    )(dstd)


CH = 40


def _scatter_body(gflat, srcp, dstp, out, src_v, dst_v, buf0, buf1, acc,
                  sem0, sem1, *, nb):
    c = lax.axis_index("c")
    s = lax.axis_index("s")
    w = c * NS + s
    _zero_rows(buf0, 128, BATCH)
    _zero_acc_stripe(acc, buf0, s * STRIPE)
    plsc.subcore_barrier()

    def step(b, buf, sem, start_next):
        pltpu.make_async_copy(gflat.at[src_v.at[b]], buf, sem).wait()
        pltpu.sync_copy(buf, acc.at[dst_v.at[b]], add=True)
        if start_next:
            pltpu.async_copy(gflat.at[src_v.at[b + 2]], buf, sem)

    def chunk(q, _):
        pltpu.sync_copy(srcp.at[w, pl.ds(q * CH, CH)], src_v)
        pltpu.sync_copy(dstp.at[w, pl.ds(q * CH, CH)], dst_v)
        pltpu.async_copy(gflat.at[src_v.at[0]], buf0, sem0)
        pltpu.async_copy(gflat.at[src_v.at[1]], buf1, sem1)

        def body(j2, _):
            step(2 * j2, buf0, sem0, True)
            step(2 * j2 + 1, buf1, sem1, True)
            return 0
        lax.fori_loop(0, CH // 2 - 1, body, 0)
        step(CH - 2, buf0, sem0, False)
        step(CH - 1, buf1, sem1, False)
        return 0
    lax.fori_loop(0, nb // CH, chunk, 0)
    plsc.subcore_barrier()
    pltpu.sync_copy(acc.at[pl.ds(s * STRIPE, STRIPE)],
                    out.at[pl.ds(c * NT + s * STRIPE, STRIPE)])


def _edge_scatter(gflat, srcp, dstp, nb):
    return pl.kernel(
        functools.partial(_scatter_body, nb=nb),
        out_type=jax.ShapeDtypeStruct((NC * NT, BATCH), jnp.float32),
        mesh=plsc.VectorSubcoreMesh(core_axis_name="c", subcore_axis_name="s"),
        scratch_types=[
            pltpu.VMEM((CH, BATCH), jnp.int32),
            pltpu.VMEM((CH, BATCH), jnp.int32),
            pltpu.VMEM((BATCH, BATCH), jnp.float32),
            pltpu.VMEM((BATCH, BATCH), jnp.float32),
            pltpu.VMEM_SHARED((NT, BATCH), jnp.float32),
            pltpu.SemaphoreType.DMA,
            pltpu.SemaphoreType.DMA,
        ],
    )(gflat, srcp, dstp)


def _dinv_of(deg_ref):
    dvec = deg_ref[0, :, 0:1] + deg_ref[1, :, 0:1] + jnp.float32(1)
    return lax.rsqrt(dvec)


def _t0_body(x_ref, w_ref, deg_ref, g_ref):
    dinv = _dinv_of(deg_ref)
    p = jnp.dot(x_ref[...], w_ref[...], preferred_element_type=jnp.float32)
    g = dinv * p
    hc = g.shape[1] // 2
    g_ref[0] = g[:, :hc]
    g_ref[1] = g[:, hc:]


def _mid_body(s_ref, g_ref, deg_ref, b_ref, w_ref, h_ref, gn_ref, *, split):
    dinv = _dinv_of(deg_ref)
    t = jnp.concatenate([s_ref[0] + g_ref[0], s_ref[1] + g_ref[1]], axis=1)
    h = jnp.tanh(dinv * t + b_ref[...])
    h_ref[...] = h
    p = jnp.dot(h, w_ref[...], preferred_element_type=jnp.float32)
    gn = dinv * p
    if split:
        hc = gn.shape[1] // 2
        gn_ref[0] = gn[:, :hc]
        gn_ref[1] = gn[:, hc:]
    else:
        gn_ref[...] = gn


def _fin_body(s_ref, g_ref, deg_ref, b_ref, o_ref):
    dinv = _dinv_of(deg_ref)
    t = s_ref[0] + s_ref[1] + g_ref[...]
    o_ref[...] = jnp.tanh(dinv * t + b_ref[...])


def _row_blk(shape):
    return pl.BlockSpec((1,) * (len(shape) - 2) + (BN, shape[-1]),
                        lambda i: (0,) * (len(shape) - 2) + (i, 0))


def _full_blk(shape):
    return pl.BlockSpec(shape, lambda i: (0,) * len(shape))


def _t0(x, w0, deg2):
    return pl.pallas_call(
        _t0_body,
        grid=(N // BN,),
        in_specs=[_row_blk((N, H)), _full_blk((H, H)),
                  pl.BlockSpec((2, BN, 16), lambda i: (0, i, 0))],
        out_specs=pl.BlockSpec((2, BN, H // 2), lambda i: (0, i, 0)),
        out_shape=jax.ShapeDtypeStruct((2, N, H // 2), jnp.float32),
    )(x, w0, deg2)


def _tmid(s2, g2, deg2, b, w, split):
    k = w.shape[1]
    if split:
        gn_spec = pl.BlockSpec((2, BN, k // 2), lambda i: (0, i, 0))
        gn_shape = jax.ShapeDtypeStruct((2, N, k // 2), jnp.float32)
    else:
        gn_spec = _row_blk((N, k))
        gn_shape = jax.ShapeDtypeStruct((N, k), jnp.float32)
    return pl.pallas_call(
        functools.partial(_mid_body, split=split),
        grid=(N // BN,),
        in_specs=[pl.BlockSpec((2, BN, H // 2), lambda i: (0, i, 0)),
                  pl.BlockSpec((2, BN, H // 2), lambda i: (0, i, 0)),
                  pl.BlockSpec((2, BN, 16), lambda i: (0, i, 0)),
                  _full_blk((1, H)), _full_blk((H, k))],
        out_specs=[_row_blk((N, H)), gn_spec],
        out_shape=[jax.ShapeDtypeStruct((N, H), jnp.float32), gn_shape],
    )(s2, g2, deg2, b, w)


def _tfin(s2, g2, deg2, b):
    return pl.pallas_call(
        _fin_body,
        grid=(N // BN,),
        in_specs=[pl.BlockSpec((2, BN, A), lambda i: (0, i, 0)),
                  _row_blk((N, A)),
                  pl.BlockSpec((2, BN, 16), lambda i: (0, i, 0)),
                  _full_blk((1, A))],
        out_specs=_row_blk((N, A)),
        out_shape=jax.ShapeDtypeStruct((N, A), jnp.float32),
    )(s2, g2, deg2, b)


def kernel(x, edge_index, W0, b0, W1, b1, W2, b2):
    e = edge_index.shape[1]
    src, dst = edge_index[0], edge_index[1]
    padidx = jnp.arange(EPAD - e, dtype=jnp.int32) % 16
    srcf = jnp.concatenate([src, padidx])
    dstf = jnp.concatenate([dst, N + padidx])
    src01 = jnp.stack([srcf, srcf + N]).reshape(NC * NS, NB, BATCH)
    dst01 = jnp.stack([dstf, dstf]).reshape(NC * NS, NB, BATCH)
    src2 = srcf.reshape(NC * NS, NBD, BATCH)
    dst2 = dstf.reshape(NC * NS, NBD, BATCH)

    deg2 = _deg_counts(dst2).reshape(NC, NT, 16)

    def layer_scatter01(gsplit):
        sflat = _edge_scatter(gsplit.reshape(NC * N, BATCH), src01, dst01, NB)
        return sflat.reshape(NC, NT, BATCH)

    g0 = _t0(x, W0, deg2)
    s0 = layer_scatter01(g0)
    h1, g1 = _tmid(s0, g0, deg2, b0.reshape(1, H), W1, split=True)
    s1 = layer_scatter01(g1)
    h2, g2 = _tmid(s1, g1, deg2, b1.reshape(1, H), W2, split=False)
    s2 = _edge_scatter(g2, src2, dst2, NBD).reshape(NC, NT, A)
    out = _tfin(s2, g2, deg2, b2.reshape(1, A))
    return (out, h1, h2)


# speedup vs baseline: 1.0227x; 1.0104x over previous
import functools

import jax
import jax.numpy as jnp
from jax import lax
from jax.experimental import pallas as pl
from jax.experimental.pallas import tpu as pltpu
from jax.experimental.pallas import tpu_sc as plsc

N = 10000
H = 256
A = 128
NC = 2
NS = 16
BATCH = 128
NT = N + 112
STRIPE = NT // NS
EPAD = 163840
NB = EPAD // (NS * BATCH)
NBD = EPAD // (NC * NS * BATCH)
BN = 2000


def _zero_rows(buf, rows, cols):
    def zrow(r, _):
        for k in range(cols // 16):
            buf[r, pl.ds(k * 16, 16)] = jnp.zeros((16,), jnp.float32)
        return 0
    lax.fori_loop(0, rows, zrow, 0)


def _zero_acc_stripe(acc, zbuf, base):
    full, rem = STRIPE // 128, STRIPE % 128
    for k in range(full):
        pltpu.sync_copy(zbuf, acc.at[pl.ds(base + k * 128, 128)])
    if rem:
        pltpu.sync_copy(zbuf.at[pl.ds(0, rem)],
                        acc.at[pl.ds(base + full * 128, rem)])


def _deg_body(dstd, out, dst_v, e0, zbuf, acc, semd):
    c = lax.axis_index("c")
    s = lax.axis_index("s")
    w = c * NS + s
    pltpu.sync_copy(dstd.at[w], dst_v)
    _zero_rows(zbuf, 128, 16)
    v = jnp.where(lax.iota(jnp.int32, 16) == 0, jnp.float32(1), jnp.float32(0))

    def erow(r, _):
        e0[r, :] = v
        return 0
    lax.fori_loop(0, 128, erow, 0)
    _zero_acc_stripe(acc, zbuf, s * STRIPE)
    plsc.subcore_barrier()

    def body(j, _):
        pltpu.sync_copy(e0, acc.at[dst_v.at[j]], add=True)
        return 0
    lax.fori_loop(0, NBD, body, 0)
    plsc.subcore_barrier()
    pltpu.sync_copy(acc.at[pl.ds(s * STRIPE, STRIPE)],
                    out.at[pl.ds(c * NT + s * STRIPE, STRIPE)])


def _deg_counts(dstd):
    return pl.kernel(
        _deg_body,
        out_type=jax.ShapeDtypeStruct((NC * NT, 16), jnp.float32),
        mesh=plsc.VectorSubcoreMesh(core_axis_name="c", subcore_axis_name="s"),
        scratch_types=[
            pltpu.VMEM((NBD, BATCH), jnp.int32),
            pltpu.VMEM((128, 16), jnp.float32),
            pltpu.VMEM((128, 16), jnp.float32),
            pltpu.VMEM_SHARED((NT, 16), jnp.float32),
            pltpu.SemaphoreType.DMA,
        ],
    )(dstd)


def _scatter_body(gflat, srcp, dstp, out, *rest, nb, ch, wdiv, toff, dbl):
    if dbl:
        sv0, dv0, sv1, dv1, buf0, buf1, acc, sem0, sem1, semi = rest
    else:
        sv0, dv0, buf0, buf1, acc, sem0, sem1 = rest
        sv1 = dv1 = semi = None
    c = lax.axis_index("c")
    s = lax.axis_index("s")
    w = c * NS + s
    r = lax.rem(w, jnp.int32(wdiv))
    tab = gflat.at[pl.ds(c * toff, N)] if toff else gflat
    nch = nb // ch
    _zero_rows(buf0, 128, BATCH)
    _zero_acc_stripe(acc, buf0, s * STRIPE)
    plsc.subcore_barrier()
    pltpu.sync_copy(srcp.at[r, pl.ds(0, ch)], sv0)
    pltpu.sync_copy(dstp.at[r, pl.ds(0, ch)], dv0)
    pltpu.async_copy(tab.at[sv0.at[0]], buf0, sem0)
    pltpu.async_copy(tab.at[sv0.at[1]], buf1, sem1)
    idx = [(sv0, dv0), (sv1, dv1)]
    for q in range(nch):
        sv, dv = idx[q % 2]
        has_next = q + 1 < nch
        nsv, ndv = idx[(q + 1) % 2] if has_next else (None, None)
        if has_next:
            pltpu.async_copy(srcp.at[r, pl.ds((q + 1) * ch, ch)], nsv, semi)
            pltpu.async_copy(dstp.at[r, pl.ds((q + 1) * ch, ch)], ndv, semi)

        def body(j2, _, sv=sv, dv=dv):
            for t in range(2):
                b = 2 * j2 + t
                buf, sem = (buf0, sem0) if t == 0 else (buf1, sem1)
                pltpu.make_async_copy(tab.at[sv.at[b]], buf, sem).wait()
                pltpu.sync_copy(buf, acc.at[dv.at[b]], add=True)
                pltpu.async_copy(tab.at[sv.at[b + 2]], buf, sem)
            return 0
        lax.fori_loop(0, ch // 2 - 1, body, 0)
        if has_next:
            pltpu.make_async_copy(
                srcp.at[r, pl.ds((q + 1) * ch, ch)], nsv, semi).wait()
            pltpu.make_async_copy(
                dstp.at[r, pl.ds((q + 1) * ch, ch)], ndv, semi).wait()
        for t in range(2):
            b = ch - 2 + t
            buf, sem = (buf0, sem0) if t == 0 else (buf1, sem1)
            pltpu.make_async_copy(tab.at[sv.at[b]], buf, sem).wait()
            pltpu.sync_copy(buf, acc.at[dv.at[b]], add=True)
            if has_next:
                pltpu.async_copy(tab.at[nsv.at[t]], buf, sem)
    plsc.subcore_barrier()
    pltpu.sync_copy(acc.at[pl.ds(s * STRIPE, STRIPE)],
                    out.at[pl.ds(c * NT + s * STRIPE, STRIPE)])


def _edge_scatter(gflat, srcp, dstp, nb, ch, wdiv, toff):
    dbl = nb // ch > 1
    idx_scr = [pltpu.VMEM((ch, BATCH), jnp.int32)] * (4 if dbl else 2)
    sems = [pltpu.SemaphoreType.DMA] * (3 if dbl else 2)
    return pl.kernel(
        functools.partial(_scatter_body, nb=nb, ch=ch, wdiv=wdiv, toff=toff,
                          dbl=dbl),
        out_type=jax.ShapeDtypeStruct((NC * NT, BATCH), jnp.float32),
        mesh=plsc.VectorSubcoreMesh(core_axis_name="c", subcore_axis_name="s"),
        scratch_types=idx_scr + [
            pltpu.VMEM((BATCH, BATCH), jnp.float32),
            pltpu.VMEM((BATCH, BATCH), jnp.float32),
            pltpu.VMEM_SHARED((NT, BATCH), jnp.float32),
        ] + sems,
    )(gflat, srcp, dstp)


def _dinv_of(deg_ref):
    dvec = deg_ref[0, :, 0:1] + deg_ref[1, :, 0:1] + jnp.float32(1)
    return lax.rsqrt(dvec)


def _t0_body(x_ref, w_ref, deg_ref, g_ref):
    dinv = _dinv_of(deg_ref)
    p = jnp.dot(x_ref[...], w_ref[...], preferred_element_type=jnp.float32)
    g = dinv * p
    hc = g.shape[1] // 2
    g_ref[0] = g[:, :hc]
    g_ref[1] = g[:, hc:]


def _mid_body(s_ref, g_ref, deg_ref, b_ref, w_ref, h_ref, gn_ref, *, split):
    dinv = _dinv_of(deg_ref)
    t = jnp.concatenate([s_ref[0] + g_ref[0], s_ref[1] + g_ref[1]], axis=1)
    h = jnp.tanh(dinv * t + b_ref[...])
    h_ref[...] = h
    p = jnp.dot(h, w_ref[...], preferred_element_type=jnp.float32)
    gn = dinv * p
    if split:
        hc = gn.shape[1] // 2
        gn_ref[0] = gn[:, :hc]
        gn_ref[1] = gn[:, hc:]
    else:
        gn_ref[...] = gn


def _fin_body(s_ref, g_ref, deg_ref, b_ref, o_ref):
    dinv = _dinv_of(deg_ref)
    t = s_ref[0] + s_ref[1] + g_ref[...]
    o_ref[...] = jnp.tanh(dinv * t + b_ref[...])


def _row_blk(shape):
    return pl.BlockSpec((1,) * (len(shape) - 2) + (BN, shape[-1]),
                        lambda i: (0,) * (len(shape) - 2) + (i, 0))


def _full_blk(shape):
    return pl.BlockSpec(shape, lambda i: (0,) * len(shape))


def _t0(x, w0, deg2):
    return pl.pallas_call(
        _t0_body,
        grid=(N // BN,),
        in_specs=[_row_blk((N, H)), _full_blk((H, H)),
                  pl.BlockSpec((2, BN, 16), lambda i: (0, i, 0))],
        out_specs=pl.BlockSpec((2, BN, H // 2), lambda i: (0, i, 0)),
        out_shape=jax.ShapeDtypeStruct((2, N, H // 2), jnp.float32),
    )(x, w0, deg2)


def _tmid(s2, g2, deg2, b, w, split):
    k = w.shape[1]
    if split:
        gn_spec = pl.BlockSpec((2, BN, k // 2), lambda i: (0, i, 0))
        gn_shape = jax.ShapeDtypeStruct((2, N, k // 2), jnp.float32)
    else:
        gn_spec = _row_blk((N, k))
        gn_shape = jax.ShapeDtypeStruct((N, k), jnp.float32)
    return pl.pallas_call(
        functools.partial(_mid_body, split=split),
        grid=(N // BN,),
        in_specs=[pl.BlockSpec((2, BN, H // 2), lambda i: (0, i, 0)),
                  pl.BlockSpec((2, BN, H // 2), lambda i: (0, i, 0)),
                  pl.BlockSpec((2, BN, 16), lambda i: (0, i, 0)),
                  _full_blk((1, H)), _full_blk((H, k))],
        out_specs=[_row_blk((N, H)), gn_spec],
        out_shape=[jax.ShapeDtypeStruct((N, H), jnp.float32), gn_shape],
    )(s2, g2, deg2, b, w)


def _tfin(s2, g2, deg2, b):
    return pl.pallas_call(
        _fin_body,
        grid=(N // BN,),
        in_specs=[pl.BlockSpec((2, BN, A), lambda i: (0, i, 0)),
                  _row_blk((N, A)),
                  pl.BlockSpec((2, BN, 16), lambda i: (0, i, 0)),
                  _full_blk((1, A))],
        out_specs=_row_blk((N, A)),
        out_shape=jax.ShapeDtypeStruct((N, A), jnp.float32),
    )(s2, g2, deg2, b)


def kernel(x, edge_index, W0, b0, W1, b1, W2, b2):
    e = edge_index.shape[1]
    src, dst = edge_index[0], edge_index[1]
    padidx = jnp.arange(EPAD - e, dtype=jnp.int32) % 16
    srcf = jnp.concatenate([src, padidx])
    dstf = jnp.concatenate([dst, N + padidx])
    src01 = jnp.stack([srcf, srcf + N]).reshape(NC * NS, NB, BATCH)
    dst01 = jnp.stack([dstf, dstf]).reshape(NC * NS, NB, BATCH)
    src2 = srcf.reshape(NC * NS, NBD, BATCH)
    dst2 = dstf.reshape(NC * NS, NBD, BATCH)

    deg2 = _deg_counts(dst2).reshape(NC, NT, 16)

    def layer_scatter01(gsplit):
        sflat = _edge_scatter(gsplit.reshape(NC * N, BATCH), src01, dst01,
                              NB, 16, NC * NS, 0)
        return sflat.reshape(NC, NT, BATCH)

    g0 = _t0(x, W0, deg2)
    s0 = layer_scatter01(g0)
    h1, g1 = _tmid(s0, g0, deg2, b0.reshape(1, H), W1, split=True)
    s1 = layer_scatter01(g1)
    h2, g2 = _tmid(s1, g1, deg2, b1.reshape(1, H), W2, split=False)
    s2 = _edge_scatter(g2, src2, dst2, NBD, NBD, NC * NS, 0)
    s2 = s2.reshape(NC, NT, A)
    out = _tfin(s2, g2, deg2, b2.reshape(1, A))
    return (out, h1, h2)
